# baseline (speedup 1.0000x reference)
import functools

import jax
import jax.numpy as jnp
from jax import lax
from jax.experimental import pallas as pl
from jax.experimental.pallas import tpu as pltpu
from jax.experimental.pallas import tpu_sc as plsc

L = 16
NT = 16
NPAD = 10240
BAND = NPAD // NT
FN = 4 * NPAD


def _tc_in(xTp, W1):
    n = xTp.shape[1]
    F = W1.shape[0]

    def body(x_ref, w_ref, h_ref):
        h_ref[...] = lax.dot_general(w_ref[...], x_ref[...],
                                     (((1,), (0,)), ((), ())),
                                     preferred_element_type=jnp.float32)

    return pl.pallas_call(
        body,
        out_shape=jax.ShapeDtypeStruct((F, n), jnp.float32),
    )(xTp, W1)


def _tanh(z):
    ez = jnp.exp(2.0 * z)
    return 1.0 - 2.0 / (ez + 1.0)


def _rsqrt_newton(x):
    i = plsc.bitcast(x, jnp.int32)
    i = jnp.int32(0x5F3759DF) - lax.shift_right_logical(i, 1)
    y = plsc.bitcast(i, jnp.float32)
    for _ in range(3):
        y = y * (1.5 - 0.5 * x * y * y)
    return y


@functools.cache
def _mega_kernel(e: int):
    ept = e // NT
    echunk = ept // 5
    assert echunk % L == 0
    mesh = plsc.VectorSubcoreMesh(core_axis_name="c", subcore_axis_name="s")

    @functools.partial(
        pl.kernel,
        out_type=(jax.ShapeDtypeStruct((10 * NPAD,), jnp.float32),
                  jax.ShapeDtypeStruct((2 * NPAD,), jnp.float32)),
        mesh=mesh,
        scratch_types=[
            pltpu.VMEM((FN,), jnp.float32),
            pltpu.VMEM((FN,), jnp.float32),
            pltpu.VMEM((echunk,), jnp.int32),
            pltpu.VMEM((echunk,), jnp.int32),
            pltpu.VMEM((BAND,), jnp.float32),
            pltpu.VMEM((4 * BAND,), jnp.float32),
            pltpu.VMEM((4 * BAND,), jnp.float32),
            pltpu.VMEM((8 * BAND,), jnp.float32),
            pltpu.VMEM((64 * L,), jnp.float32),
            pltpu.SemaphoreType.DMA,
            pltpu.VMEM_SHARED((NT * 2 * NPAD,), jnp.float32),
            pltpu.VMEM_SHARED((FN,), jnp.float32),
        ],
        compiler_params=pltpu.CompilerParams(needs_layout_passes=False),
    )
    def mega(h1_hbm, src_hbm, dst_hbm, wc_hbm, out_hbm, h_hbm,
             g_v, s_v, src_v, dst_v, dis_v, sacc_v, a_v, tmp_v, wc_v, sem,
             slots_sh, g_sh):
        cid = lax.axis_index("c")
        sid = lax.axis_index("s")
        ebase = sid * ept
        zeros = jnp.zeros((L,), jnp.float32)
        ones = jnp.ones((L,), jnp.float32)

        def wrow(r):
            return wc_v[pl.ds(r * L, L)]

        def zero_sv(nwords):
            @plsc.parallel_loop(0, nwords // L, unroll=8)
            def _(i):
                s_v[pl.ds(i * L, L)] = zeros

        def edge_loop(F, use_src):
            for c in range(5):
                if use_src:
                    pltpu.sync_copy(
                        src_hbm.at[pl.ds(ebase + c * echunk, echunk)], src_v)
                pltpu.sync_copy(
                    dst_hbm.at[pl.ds(ebase + c * echunk, echunk)], dst_v)

                @plsc.parallel_loop(0, echunk // L, unroll=4)
                def _(i):
                    dv = dst_v[pl.ds(i * L, L)]
                    if use_src:
                        sv = src_v[pl.ds(i * L, L)]
                        for f in range(F):
                            vals = plsc.load_gather(
                                g_v, [sv + jnp.int32(f * NPAD)])
                            plsc.addupdate_scatter(
                                s_v, [dv + jnp.int32(f * NPAD)], vals)
                    else:
                        plsc.addupdate_scatter(s_v, [dv], ones)

        def publish(pair, nf):
            pltpu.sync_copy(
                s_v.at[pl.ds(pair * 2 * NPAD, nf * NPAD)],
                slots_sh.at[pl.ds(sid * 2 * NPAD, nf * NPAD)])

        def gather_band(fl, dst_off):
            for half in range(2):
                descs = [
                    pltpu.async_copy(
                        slots_sh.at[pl.ds((half * 8 + kk) * 2 * NPAD
                                          + fl * NPAD + sid * BAND, BAND)],
                        tmp_v.at[pl.ds(kk * BAND, BAND)], sem)
                    for kk in range(8)
                ]
                for dsc in descs:
                    dsc.wait()

                if half == 0:
                    @plsc.parallel_loop(0, BAND // L, unroll=2)
                    def _(j):
                        acc = tmp_v[pl.ds(j * L, L)]
                        for kk in range(1, 8):
                            acc = acc + tmp_v[pl.ds(kk * BAND + j * L, L)]
                        sacc_v[pl.ds(dst_off + j * L, L)] = acc
                else:
                    @plsc.parallel_loop(0, BAND // L, unroll=2)
                    def _(j):
                        acc = tmp_v[pl.ds(j * L, L)]
                        for kk in range(1, 8):
                            acc = acc + tmp_v[pl.ds(kk * BAND + j * L, L)]
                        sacc_v[pl.ds(dst_off + j * L, L)] = \
                            sacc_v[pl.ds(dst_off + j * L, L)] + acc

        pltpu.sync_copy(wc_hbm, wc_v)
        zero_sv(NPAD)

        edge_loop(1, use_src=False)
        publish(0, 1)
        plsc.subcore_barrier()

        gather_band(0, 0)
        plsc.subcore_barrier()

        @plsc.parallel_loop(0, BAND // L, unroll=4)
        def _(j):
            deg = sacc_v[pl.ds(j * L, L)] + 1.0
            dis_v[pl.ds(j * L, L)] = _rsqrt_newton(deg)

        for f in range(4):
            pltpu.sync_copy(
                h1_hbm.at[pl.ds(f * NPAD + sid * BAND, BAND)],
                a_v.at[pl.ds(f * BAND, BAND)])

        @plsc.parallel_loop(0, BAND // L, unroll=4)
        def _(j):
            d = dis_v[pl.ds(j * L, L)]
            for f in range(4):
                a_v[pl.ds(f * BAND + j * L, L)] = \
                    a_v[pl.ds(f * BAND + j * L, L)] * d

        for f in range(4):
            pltpu.sync_copy(a_v.at[pl.ds(f * BAND, BAND)],
                            g_sh.at[pl.ds(f * NPAD + sid * BAND, BAND)])
        zero_sv(NPAD)
        plsc.subcore_barrier()

        layers = [(4, 4, 0, 44), (4, 2, 16, 48), (2, 2, None, 52)]
        for li, (Fi, Fo, wr0, br0) in enumerate(layers):
            pltpu.sync_copy(g_sh.at[pl.ds(0, Fi * NPAD)],
                            g_v.at[pl.ds(0, Fi * NPAD)])

            edge_loop(Fi, use_src=True)
            for pair in range((Fi + 1) // 2):
                nf = min(2, Fi - 2 * pair)
                publish(pair, nf)
                plsc.subcore_barrier()
                for k in range(nf):
                    gather_band(k, (2 * pair + k) * BAND)
                plsc.subcore_barrier()

            bs = [wrow(br0 + f) for f in range(Fi)]
            if li < 2:
                wm = [[wrow(wr0 + fo * Fi + f) for f in range(Fi)]
                      for fo in range(Fo)]

            @plsc.parallel_loop(0, BAND // L, unroll=2)
            def _(j):
                d = dis_v[pl.ds(j * L, L)]
                acts = []
                for f in range(Fi):
                    gb = g_v[pl.ds(f * NPAD + sid * BAND + j * L, L)]
                    z = d * (sacc_v[pl.ds(f * BAND + j * L, L)] + gb) + bs[f]
                    acts.append(_tanh(z))
                if li < 2:
                    for fo in range(Fo):
                        acc = acts[0] * wm[fo][0]
                        for f in range(1, Fi):
                            acc = acc + acts[f] * wm[fo][f]
                        a_v[pl.ds(fo * BAND + j * L, L)] = acc * d
                else:
                    for f in range(Fi):
                        a_v[pl.ds(f * BAND + j * L, L)] = acts[f]

            if li < 2:
                for fo in range(Fo):
                    pltpu.sync_copy(
                        a_v.at[pl.ds(fo * BAND, BAND)],
                        g_sh.at[pl.ds(fo * NPAD + sid * BAND, BAND)])
                zero_sv(max(Fi, Fo) * NPAD)
                plsc.subcore_barrier()

        @pl.when(cid == 0)
        def _():
            for f in range(2):
                pltpu.sync_copy(a_v.at[pl.ds(f * BAND, BAND)],
                                h_hbm.at[pl.ds(f * NPAD + sid * BAND, BAND)])

        for grp in range(3):
            c0 = grp * 4
            ncc = min(4, 10 - c0)
            wgs = [(wrow(24 + cc * 2), wrow(24 + cc * 2 + 1), wrow(54 + cc))
                   for cc in range(c0, c0 + ncc)]

            @plsc.parallel_loop(0, BAND // L, unroll=2)
            def _(j):
                a0 = a_v[pl.ds(j * L, L)]
                a1 = a_v[pl.ds(BAND + j * L, L)]
                for k, (w0, w1, bcc) in enumerate(wgs):
                    sacc_v[pl.ds(k * BAND + j * L, L)] = \
                        a0 * w0 + a1 * w1 + bcc

            @pl.when(cid == 0)
            def _():
                for k in range(ncc):
                    pltpu.sync_copy(
                        sacc_v.at[pl.ds(k * BAND, BAND)],
                        out_hbm.at[pl.ds((c0 + k) * NPAD + sid * BAND, BAND)])

    return mega


def _build_wconst(W2, b1, W3, b2, Wc, b3, bc):
    rows = jnp.concatenate([
        W2.reshape(-1), W3.reshape(-1), Wc.reshape(-1),
        b1, b2, b3, bc,
    ])
    return jnp.broadcast_to(rows[:, None], (64, L)).reshape(-1)


@jax.jit
def kernel(x, edge_index, W1, b1, W2, b2, W3, b3, Wc, bc):
    n, d = x.shape
    e = edge_index.shape[1]
    assert n <= NPAD and e % (NT * 2 * L) == 0

    src = edge_index[0]
    dst = edge_index[1]
    xTp = jnp.pad(x.T, ((0, 0), (0, NPAD - n)))
    h1raw = _tc_in(xTp, W1).reshape(-1)
    wconst = _build_wconst(W2, b1, W3, b2, Wc, b3, bc)

    out_f, h_f = _mega_kernel(e)(h1raw, src, dst, wconst)
    out = out_f.reshape(10, NPAD)[:, :n].T
    h = h_f.reshape(2, NPAD)[:, :n].T
    return out, h
